# Initial kernel scaffold; baseline (speedup 1.0000x reference)
import jax
import jax.numpy as jnp
from jax.experimental import pallas as pl


def kernel(x, W1, b1, W2, b2, Ws1, bs1, Ws2, bs2, p1, p2, edge_index):
    raise NotImplementedError("write your pallas kernel here")


# speedup vs baseline: 16.5233x; 16.5233x over previous
import functools

import jax
import jax.numpy as jnp
from jax import lax
from jax.experimental import pallas as pl
from jax.experimental.pallas import tpu as pltpu
from jax.experimental.pallas import tpu_sc as plsc

_N = 10000
_E = 320000
_D = 128
_WID = 128
_C = 64

_NC = 2
_NS = 16
_NW = _NC * _NS
_LANES = 16
_CH = 128
_KCH = -(-_E // (_NW * _CH))
_EPW = _KCH * _CH
_EPAD = _EPW * _NW
_NPAD = 10240
_RPT = _NPAD // _NS
_BN = 1280
_GRID = _NPAD // _BN


def _sc_mesh():
    return plsc.VectorSubcoreMesh(
        core_axis_name="c", subcore_axis_name="s",
        num_cores=_NC, num_subcores=_NS)


@functools.cache
def _build_sc_degree():
    @functools.partial(
        pl.kernel,
        out_type=jax.ShapeDtypeStruct((_NW, _NPAD), jnp.float32),
        mesh=_sc_mesh(),
        scratch_types=[
            pltpu.VMEM((_EPW,), jnp.int32),
            pltpu.VMEM((_NPAD,), jnp.float32),
        ],
        compiler_params=pltpu.CompilerParams(needs_layout_passes=False, use_tc_tiling_on_sc=False),
    )
    def _sc_degree(dst_hbm, out_hbm, idx_v, deg_v):
        cid = lax.axis_index("c")
        sid = lax.axis_index("s")
        wid = sid * _NC + cid
        pltpu.sync_copy(dst_hbm.at[wid], idx_v)
        zeros = jnp.zeros((_LANES,), jnp.float32)

        def zero_body(i, carry):
            deg_v[pl.ds(i * _LANES, _LANES)] = zeros
            return carry

        lax.fori_loop(0, _NPAD // _LANES, zero_body, 0)
        ones = jnp.ones((_LANES,), jnp.float32)

        def body(g, carry):
            idx = idx_v[pl.ds(g * _LANES, _LANES)]
            plsc.addupdate_scatter(deg_v, [idx], ones)
            return carry

        lax.fori_loop(0, _EPW // _LANES, body, 0)
        pltpu.sync_copy(deg_v, out_hbm.at[wid])

    return _sc_degree


@functools.cache
def _build_edge_scatter(w):
    @functools.partial(
        pl.kernel,
        out_type=jax.ShapeDtypeStruct((_NC, _NPAD, w), jnp.float32),
        mesh=_sc_mesh(),
        scratch_types=[
            pltpu.VMEM((_KCH, _CH), jnp.int32),
            pltpu.VMEM((_KCH, _CH), jnp.int32),
            pltpu.VMEM((_CH, w), jnp.float32),
            pltpu.VMEM_SHARED((_NPAD, w), jnp.float32),
            pltpu.SemaphoreType.DMA,
        ],
        compiler_params=pltpu.CompilerParams(needs_layout_passes=False, use_tc_tiling_on_sc=False),
    )
    def _scat(g_hbm, src_hbm, dst_hbm, z_hbm, out_hbm,
              src_v, dst_v, rows_v, acc_sh, sem):
        cid = lax.axis_index("c")
        sid = lax.axis_index("s")
        wid = sid * _NC + cid
        pltpu.sync_copy(z_hbm.at[pl.ds(sid * _RPT, _RPT)],
                        acc_sh.at[pl.ds(sid * _RPT, _RPT)])
        pltpu.sync_copy(src_hbm.at[wid], src_v)
        pltpu.sync_copy(dst_hbm.at[wid], dst_v)
        plsc.subcore_barrier()

        def body(j, carry):
            pltpu.async_copy(g_hbm.at[src_v.at[j]], rows_v, sem).wait()
            pltpu.sync_copy(rows_v, acc_sh.at[dst_v.at[j]], add=True)
            return carry

        lax.fori_loop(0, _KCH, body, 0)
        plsc.subcore_barrier()
        pltpu.sync_copy(acc_sh.at[pl.ds(sid * _RPT, _RPT)],
                        out_hbm.at[cid, pl.ds(sid * _RPT, _RPT)])

    return _scat


def _tc_stage1(degp, x, w1):
    def body(degp_ref, x_ref, w1_ref, dinv_ref, g1_ref):
        deg = jnp.sum(degp_ref[...], axis=0) + 1.0
        dinv = lax.rsqrt(deg)[:, None]
        dinv_ref[...] = dinv
        h = jnp.dot(x_ref[...], w1_ref[...], preferred_element_type=jnp.float32)
        g1_ref[...] = h * dinv

    return pl.pallas_call(
        body,
        grid=(_GRID,),
        in_specs=[
            pl.BlockSpec((_NW, _BN), lambda i: (0, i)),
            pl.BlockSpec((_BN, _D), lambda i: (i, 0)),
            pl.BlockSpec((_D, _WID), lambda i: (0, 0)),
        ],
        out_specs=[
            pl.BlockSpec((_BN, 1), lambda i: (i, 0)),
            pl.BlockSpec((_BN, _WID), lambda i: (i, 0)),
        ],
        out_shape=[
            jax.ShapeDtypeStruct((_NPAD, 1), jnp.float32),
            jax.ShapeDtypeStruct((_NPAD, _WID), jnp.float32),
        ],
    )(degp, x, w1)


def _tc_stage2(s1p, g1, dinv, x, ws1, bs1, b1, w2, ws2, bs2, b2, pg):
    def body(s1p_ref, g1_ref, dinv_ref, x_ref, ws1_ref, bs1_ref, b1_ref,
             w2_ref, ws2_ref, bs2_ref, b2_ref, pg_ref, g2_ref, t_ref):
        dinv = dinv_ref[...]
        s1 = s1p_ref[0] + s1p_ref[1]
        a1 = dinv * (s1 + g1_ref[...]) + b1_ref[...]
        sg1 = jax.nn.sigmoid(pg_ref[0, 0])
        sg2 = jax.nn.sigmoid(pg_ref[0, 1])
        h = jnp.maximum(a1, 0.0) + sg1 * (
            jnp.dot(x_ref[...], ws1_ref[...],
                    preferred_element_type=jnp.float32) + bs1_ref[...])
        g2_ref[...] = jnp.dot(h, w2_ref[...],
                              preferred_element_type=jnp.float32) * dinv
        t_ref[...] = sg2 * (jnp.dot(h, ws2_ref[...],
                                    preferred_element_type=jnp.float32)
                            + bs2_ref[...]) + b2_ref[...]

    return pl.pallas_call(
        body,
        grid=(_GRID,),
        in_specs=[
            pl.BlockSpec((_NC, _BN, _WID), lambda i: (0, i, 0)),
            pl.BlockSpec((_BN, _WID), lambda i: (i, 0)),
            pl.BlockSpec((_BN, 1), lambda i: (i, 0)),
            pl.BlockSpec((_BN, _D), lambda i: (i, 0)),
            pl.BlockSpec((_D, _WID), lambda i: (0, 0)),
            pl.BlockSpec((1, _WID), lambda i: (0, 0)),
            pl.BlockSpec((1, _WID), lambda i: (0, 0)),
            pl.BlockSpec((_WID, _C), lambda i: (0, 0)),
            pl.BlockSpec((_WID, _C), lambda i: (0, 0)),
            pl.BlockSpec((1, _C), lambda i: (0, 0)),
            pl.BlockSpec((1, _C), lambda i: (0, 0)),
            pl.BlockSpec((1, 2), lambda i: (0, 0)),
        ],
        out_specs=[
            pl.BlockSpec((_BN, _C), lambda i: (i, 0)),
            pl.BlockSpec((_BN, _C), lambda i: (i, 0)),
        ],
        out_shape=[
            jax.ShapeDtypeStruct((_NPAD, _C), jnp.float32),
            jax.ShapeDtypeStruct((_NPAD, _C), jnp.float32),
        ],
    )(s1p, g1, dinv, x, ws1, bs1, b1, w2, ws2, bs2, b2, pg)


def _tc_stage3(s2p, g2, dinv, t):
    def body(s2p_ref, g2_ref, dinv_ref, t_ref, out_ref):
        out_ref[...] = dinv_ref[...] * (s2p_ref[0] + s2p_ref[1]
                                        + g2_ref[...]) + t_ref[...]

    return pl.pallas_call(
        body,
        grid=(_GRID,),
        in_specs=[
            pl.BlockSpec((_NC, _BN, _C), lambda i: (0, i, 0)),
            pl.BlockSpec((_BN, _C), lambda i: (i, 0)),
            pl.BlockSpec((_BN, 1), lambda i: (i, 0)),
            pl.BlockSpec((_BN, _C), lambda i: (i, 0)),
        ],
        out_specs=pl.BlockSpec((_BN, _C), lambda i: (i, 0)),
        out_shape=jax.ShapeDtypeStruct((_NPAD, _C), jnp.float32),
    )(s2p, g2, dinv, t)


def kernel(x, W1, b1, W2, b2, Ws1, bs1, Ws2, bs2, p1, p2, edge_index):
    src = edge_index[0]
    dst = edge_index[1]
    fill = jnp.full((_EPAD - _E,), _N, dtype=jnp.int32)
    srcp = jnp.concatenate([src, fill]).reshape(_NW, _KCH, _CH)
    dstp = jnp.concatenate([dst, fill]).reshape(_NW, _KCH, _CH)
    dst_flat = dstp.reshape(_NW, _EPW)
    xpad = jnp.pad(x, ((0, _NPAD - _N), (0, 0)))
    z1 = jnp.zeros((_NPAD, _WID), jnp.float32)
    z2 = jnp.zeros((_NPAD, _C), jnp.float32)
    pg = jnp.stack([p1, p2]).reshape(1, 2)

    degp = _build_sc_degree()(dst_flat)
    dinv, g1 = _tc_stage1(degp, xpad, W1)
    s1p = _build_edge_scatter(_WID)(g1, srcp, dstp, z1)
    g2, t = _tc_stage2(s1p, g1, dinv, xpad,
                       Ws1, bs1.reshape(1, _WID), b1.reshape(1, _WID),
                       W2, Ws2, bs2.reshape(1, _C), b2.reshape(1, _C), pg)
    s2p = _build_edge_scatter(_C)(g2, srcp, dstp, z2)
    out = _tc_stage3(s2p, g2, dinv, t)
    return out[:_N]
